# baseline (speedup 1.0000x reference)
import functools

import jax
import jax.numpy as jnp
from jax import lax
from jax.experimental import pallas as pl
from jax.experimental.pallas import tpu as pltpu
from jax.experimental.pallas import tpu_sc as plsc

_LANES = 16
_NC = 2
_NS = 16
_NW = _NC * _NS
_CHUNK = 32
_NBUF = 3


@functools.lru_cache(maxsize=None)
def _build(n_rows, d_model, seq_len):
    bsz = n_rows // seq_len
    ppw = seq_len // _NW
    hb = ppw // _CHUNK
    nch = bsz * hb
    assert seq_len % (_NW * _CHUNK) == 0
    assert d_model % _LANES == 0

    mesh = plsc.VectorSubcoreMesh(core_axis_name="c", subcore_axis_name="s")

    @functools.partial(
        pl.kernel,
        mesh=mesh,
        out_type=jax.ShapeDtypeStruct((bsz, seq_len, d_model), jnp.float32),
        scratch_types=(
            [pltpu.VMEM((_NBUF, _CHUNK, d_model), jnp.float32),
             pltpu.VMEM((ppw, d_model), jnp.float32)]
            + [pltpu.VMEM((_CHUNK,), jnp.int32) for _ in range(nch)]
            + [pltpu.SemaphoreType.DMA] * 8
        ),
    )
    def emb_kernel(labels_hbm, wte_hbm, wpe_hbm, out_hbm, gbuf, wbuf, *rest):
        idxs = rest[:nch]
        gsems = rest[nch:nch + 3]
        ssems = rest[nch + 3:nch + 6]
        psem, isem = rest[nch + 6:nch + 8]
        sid = lax.axis_index("s")
        cid = lax.axis_index("c")
        wid = sid * _NC + cid
        p0 = wid * ppw

        idx_ds = [
            pltpu.async_copy(
                labels_hbm.at[pl.ds((j // hb) * seq_len + p0
                                    + (j % hb) * _CHUNK, _CHUNK)],
                idxs[j], isem)
            for j in range(nch)]
        wpe_cp = pltpu.async_copy(wpe_hbm.at[pl.ds(p0, ppw)], wbuf, psem)
        for d in idx_ds:
            d.wait()

        def start(j):
            s = j % _NBUF
            return pltpu.async_copy(wte_hbm.at[idxs[j]], gbuf.at[s], gsems[s])

        st_desc = [None] * _NBUF
        g_desc = [None] * nch
        for j in range(_NBUF - 1):
            g_desc[j] = start(j)
        for j in range(nch):
            s = j % _NBUF
            b, h = divmod(j, hb)
            g_desc[j].wait()
            if j == 0:
                wpe_cp.wait()
            if j + _NBUF - 1 < nch:
                slot = (j + _NBUF - 1) % _NBUF
                if st_desc[slot] is not None:
                    for d in st_desc[slot]:
                        d.wait()
                    st_desc[slot] = None
                g_desc[j + _NBUF - 1] = start(j + _NBUF - 1)

            @plsc.parallel_loop(0, _CHUNK, 1, unroll=8)
            def add_rows(i, _s=s, _h=h):
                for k in range(d_model // _LANES):
                    sl = pl.ds(k * _LANES, _LANES)
                    plsc.addupdate(gbuf.at[_s, i, sl], wbuf[_h * _CHUNK + i, sl])
            st_desc[s] = [pltpu.async_copy(
                gbuf.at[s],
                out_hbm.at[b, pl.ds(p0 + h * _CHUNK, _CHUNK)], ssems[s])]
        for ds_ in st_desc:
            if ds_ is not None:
                for d in ds_:
                    d.wait()

    return emb_kernel


def _copy_body(x_ref, o_ref):
    o_ref[...] = x_ref[...]


@functools.lru_cache(maxsize=None)
def _build_copy(bsz, seq_len, d_model):
    grid = (8,)
    blk = (bsz, seq_len // 8, d_model)
    return pl.pallas_call(
        _copy_body,
        grid=grid,
        in_specs=[pl.BlockSpec(blk, lambda i: (0, i, 0))],
        out_specs=pl.BlockSpec(blk, lambda i: (0, i, 0)),
        out_shape=jax.ShapeDtypeStruct((bsz, seq_len, d_model), jnp.float32),
    )


def kernel(hidden, labels, wte, wpe):
    bsz, seq_len = labels.shape
    d_model = wte.shape[1]
    n_rows = bsz * seq_len
    lab = labels.astype(jnp.int32).reshape(n_rows)
    emb = _build(n_rows, d_model, seq_len)(lab, wte, wpe)
    hidden_out = _build_copy(bsz, seq_len, d_model)(hidden)
    return (hidden_out, emb, labels)


# speedup vs baseline: 1.0044x; 1.0044x over previous
import functools

import jax
import jax.numpy as jnp
from jax import lax
from jax.experimental import pallas as pl
from jax.experimental.pallas import tpu as pltpu
from jax.experimental.pallas import tpu_sc as plsc

_LANES = 16
_NC = 2
_NS = 16
_NW = _NC * _NS
_CHUNK = 32
_NBUF = 3


@functools.lru_cache(maxsize=None)
def _build(n_rows, d_model, seq_len):
    bsz = n_rows // seq_len
    ppw = seq_len // _NW
    hb = ppw // _CHUNK
    nch = bsz * hb
    assert seq_len % (_NW * _CHUNK) == 0
    assert d_model % _LANES == 0

    mesh = plsc.VectorSubcoreMesh(core_axis_name="c", subcore_axis_name="s")

    @functools.partial(
        pl.kernel,
        mesh=mesh,
        out_type=jax.ShapeDtypeStruct((bsz, seq_len, d_model), jnp.float32),
        scratch_types=(
            [pltpu.VMEM((_NBUF, _CHUNK, d_model), jnp.float32),
             pltpu.VMEM((ppw, d_model), jnp.float32),
             pltpu.VMEM((bsz, ppw), jnp.int32)]
            + [pltpu.SemaphoreType.DMA] * 8
        ),
    )
    def emb_kernel(labels_hbm, wte_hbm, wpe_hbm, out_hbm, gbuf, wbuf, idx_v,
                   *rest):
        gsems = rest[0:3]
        ssems = rest[3:6]
        psem, isem = rest[6:8]
        sid = lax.axis_index("s")
        cid = lax.axis_index("c")
        wid = sid * _NC + cid
        p0 = wid * ppw

        wpe_cp = pltpu.async_copy(wpe_hbm.at[pl.ds(p0, ppw)], wbuf, psem)
        idx_ds = [pltpu.async_copy(labels_hbm.at[b, pl.ds(p0, ppw)],
                                   idx_v.at[b], isem) for b in range(bsz)]
        for d in idx_ds:
            d.wait()

        def start(j):
            s = j % _NBUF
            return pltpu.async_copy(
                wte_hbm.at[idx_v.at[j // hb, pl.ds((j % hb) * _CHUNK, _CHUNK)]],
                gbuf.at[s], gsems[s])

        st_desc = [None] * _NBUF
        g_desc = [None] * nch
        for j in range(_NBUF - 1):
            g_desc[j] = start(j)
        for j in range(nch):
            s = j % _NBUF
            b, h = divmod(j, hb)
            g_desc[j].wait()
            if j == 0:
                wpe_cp.wait()
            if j + _NBUF - 1 < nch:
                slot = (j + _NBUF - 1) % _NBUF
                if st_desc[slot] is not None:
                    for d in st_desc[slot]:
                        d.wait()
                    st_desc[slot] = None
                g_desc[j + _NBUF - 1] = start(j + _NBUF - 1)

            @plsc.parallel_loop(0, _CHUNK, 1, unroll=8)
            def add_rows(i, _s=s, _h=h):
                for k in range(d_model // _LANES):
                    sl = pl.ds(k * _LANES, _LANES)
                    plsc.addupdate(gbuf.at[_s, i, sl], wbuf[_h * _CHUNK + i, sl])
            st_desc[s] = [pltpu.async_copy(
                gbuf.at[s],
                out_hbm.at[b, pl.ds(p0 + h * _CHUNK, _CHUNK)], ssems[s])]
        for ds_ in st_desc:
            if ds_ is not None:
                for d in ds_:
                    d.wait()

    return emb_kernel


def _copy_body(x_ref, o_ref):
    o_ref[...] = x_ref[...]


@functools.lru_cache(maxsize=None)
def _build_copy(bsz, seq_len, d_model):
    grid = (8,)
    blk = (bsz, seq_len // 8, d_model)
    return pl.pallas_call(
        _copy_body,
        grid=grid,
        in_specs=[pl.BlockSpec(blk, lambda i: (0, i, 0))],
        out_specs=pl.BlockSpec(blk, lambda i: (0, i, 0)),
        out_shape=jax.ShapeDtypeStruct((bsz, seq_len, d_model), jnp.float32),
    )


def kernel(hidden, labels, wte, wpe):
    bsz, seq_len = labels.shape
    d_model = wte.shape[1]
    n_rows = bsz * seq_len
    lab = labels.astype(jnp.int32)
    emb = _build(n_rows, d_model, seq_len)(lab, wte, wpe)
    hidden_out = _build_copy(bsz, seq_len, d_model)(hidden)
    return (hidden_out, emb, labels)
